# Initial kernel scaffold; baseline (speedup 1.0000x reference)
import jax
import jax.numpy as jnp
from jax.experimental import pallas as pl


def kernel(x, edge_index, W1, b1, W2, b2, Wl1, bl1, Wl2, bl2):
    raise NotImplementedError("write your pallas kernel here")


# speedup vs baseline: 13.5595x; 13.5595x over previous
import functools

import jax
import jax.numpy as jnp
from jax import lax
from jax.experimental import pallas as pl
from jax.experimental.pallas import tpu as pltpu
from jax.experimental.pallas import tpu_sc as plsc

N = 10000
E = 320000
D = 128
H = 128
C = 64

NC = 2
NS = 16
NW = NC * NS
EW = E // NW
K = 80
NCHUNK = EW // K
NP = 10240
RPW = NP // NS
RCH = 128
NRCH = RPW // RCH

BN = 1024
GRID = (N + BN - 1) // BN

_mesh = plsc.VectorSubcoreMesh(
    core_axis_name="c", subcore_axis_name="s", num_cores=NC, num_subcores=NS)

_HIGH = jax.lax.Precision.HIGHEST

_SC_PARAMS = pltpu.CompilerParams(needs_layout_passes=False)


@functools.partial(
    pl.kernel,
    out_type=jax.ShapeDtypeStruct((NW, N), jnp.float32),
    mesh=_mesh,
    compiler_params=_SC_PARAMS,
    scratch_types=[
        pltpu.VMEM((EW,), jnp.int32),
        pltpu.VMEM((N,), jnp.float32),
    ],
)
def _sc_degree(dst_hbm, zeros_hbm, out_hbm, dst_v, hist_v):
    c = lax.axis_index("c")
    s = lax.axis_index("s")
    wid = c * NS + s
    pltpu.sync_copy(dst_hbm.at[wid], dst_v)
    pltpu.sync_copy(zeros_hbm, hist_v)
    ones = jnp.full((16,), 1.0, jnp.float32)

    def body(i, _):
        idx = dst_v[pl.ds(i * 16, 16)]
        plsc.addupdate_scatter(hist_v, [idx], ones)
        return _

    lax.fori_loop(0, EW // 16, body, 0, unroll=4)
    pltpu.sync_copy(hist_v, out_hbm.at[wid])


@functools.partial(
    pl.kernel,
    out_type=jax.ShapeDtypeStruct((NC, NP, H), jnp.float32),
    mesh=_mesh,
    compiler_params=_SC_PARAMS,
    scratch_types=[
        pltpu.VMEM((K,), jnp.int32),
        pltpu.VMEM((K,), jnp.int32),
        pltpu.VMEM((K, H), jnp.float32),
        pltpu.VMEM((RCH, H), jnp.float32),
        pltpu.VMEM_SHARED((NP, H), jnp.float32),
        pltpu.SemaphoreType.DMA,
    ],
)
def _sc_scatter(xs_hbm, src_hbm, dst_hbm, zrows_hbm, out_hbm,
                sidx_v, didx_v, rows_v, stage_v, acc_sh, gsem):
    c = lax.axis_index("c")
    s = lax.axis_index("s")
    wid = c * NS + s

    pltpu.sync_copy(zrows_hbm, stage_v)
    for z in range(NRCH):
        pltpu.sync_copy(stage_v, acc_sh.at[pl.ds(s * RPW + z * RCH, RCH)])
    plsc.subcore_barrier()

    def body(j, _):
        pltpu.sync_copy(src_hbm.at[wid, j], sidx_v)
        pltpu.sync_copy(dst_hbm.at[wid, j], didx_v)
        pltpu.async_copy(xs_hbm.at[sidx_v], rows_v, gsem).wait()
        pltpu.sync_copy(rows_v, acc_sh.at[didx_v], add=True)
        return _

    lax.fori_loop(0, NCHUNK, body, 0)
    plsc.subcore_barrier()

    for z in range(NRCH):
        base = s * RPW + z * RCH
        pltpu.sync_copy(acc_sh.at[pl.ds(base, RCH)], stage_v)
        pltpu.sync_copy(stage_v, out_hbm.at[c, pl.ds(base, RCH)])


def _tc_pre_body(deg_ref, x_ref, w_ref, xs_ref, dis_ref):
    deg = jnp.sum(deg_ref[...], axis=0) + 1.0
    dis = lax.rsqrt(deg)
    xw = jnp.dot(x_ref[...], w_ref[...],
                 preferred_element_type=jnp.float32, precision=_HIGH)
    xs_ref[...] = xw * dis[:, None]
    dis_ref[...] = dis


def _tc_mid_body(p_ref, xs_ref, dis_ref, b_ref, w_ref, o_ref):
    dis = dis_ref[...]
    h = (p_ref[0] + p_ref[1] + xs_ref[...]) * dis[:, None] + b_ref[...]
    h = jnp.maximum(h, 0.0)
    o_ref[...] = jnp.dot(h, w_ref[...],
                         preferred_element_type=jnp.float32,
                         precision=_HIGH) * dis[:, None]


def _tc_head_body(p_ref, xs_ref, dis_ref, b_ref, w1_ref, b1_ref, w2_ref,
                  b2_ref, o_ref):
    dis = dis_ref[...]
    h = (p_ref[0] + p_ref[1] + xs_ref[...]) * dis[:, None] + b_ref[...]
    h = jnp.maximum(h, 0.0)
    h = jnp.dot(h, w1_ref[...], preferred_element_type=jnp.float32,
                precision=_HIGH) + b1_ref[...]
    h = jnp.maximum(h, 0.0)
    lg = jnp.dot(h, w2_ref[...], preferred_element_type=jnp.float32,
                 precision=_HIGH) + b2_ref[...]
    m = jnp.max(lg, axis=-1, keepdims=True)
    lg = lg - m
    o_ref[...] = lg - jnp.log(jnp.sum(jnp.exp(lg), axis=-1, keepdims=True))


def _rows(i):
    return (i, 0)


def _full2(i):
    return (0, 0)


def _full1(i):
    return (0,)


_tc_pre = pl.pallas_call(
    _tc_pre_body,
    grid=(GRID,),
    in_specs=[
        pl.BlockSpec((NW, BN), lambda i: (0, i)),
        pl.BlockSpec((BN, D), _rows),
        pl.BlockSpec((D, H), _full2),
    ],
    out_specs=[
        pl.BlockSpec((BN, H), _rows),
        pl.BlockSpec((BN,), lambda i: (i,)),
    ],
    out_shape=[
        jax.ShapeDtypeStruct((N, H), jnp.float32),
        jax.ShapeDtypeStruct((N,), jnp.float32),
    ],
)

_tc_mid = pl.pallas_call(
    _tc_mid_body,
    grid=(GRID,),
    in_specs=[
        pl.BlockSpec((NC, BN, H), lambda i: (0, i, 0)),
        pl.BlockSpec((BN, H), _rows),
        pl.BlockSpec((BN,), lambda i: (i,)),
        pl.BlockSpec((H,), _full1),
        pl.BlockSpec((H, H), _full2),
    ],
    out_specs=pl.BlockSpec((BN, H), _rows),
    out_shape=jax.ShapeDtypeStruct((N, H), jnp.float32),
)

_tc_head = pl.pallas_call(
    _tc_head_body,
    grid=(GRID,),
    in_specs=[
        pl.BlockSpec((NC, BN, H), lambda i: (0, i, 0)),
        pl.BlockSpec((BN, H), _rows),
        pl.BlockSpec((BN,), lambda i: (i,)),
        pl.BlockSpec((H,), _full1),
        pl.BlockSpec((H, H), _full2),
        pl.BlockSpec((H,), _full1),
        pl.BlockSpec((H, C), _full2),
        pl.BlockSpec((C,), _full1),
    ],
    out_specs=pl.BlockSpec((BN, C), _rows),
    out_shape=jax.ShapeDtypeStruct((N, C), jnp.float32),
)


def kernel(x, edge_index, W1, b1, W2, b2, Wl1, bl1, Wl2, bl2):
    src_r = edge_index[0].reshape(NW, NCHUNK, K)
    dst_r = edge_index[1].reshape(NW, NCHUNK, K)
    dst_f = edge_index[1].reshape(NW, EW)
    zeros_n = jnp.zeros((N,), jnp.float32)
    zeros_rows = jnp.zeros((RCH, H), jnp.float32)

    deg_part = _sc_degree(dst_f, zeros_n)
    xs1, dis = _tc_pre(deg_part, x, W1)
    part1 = _sc_scatter(xs1, src_r, dst_r, zeros_rows)
    xs2 = _tc_mid(part1, xs1, dis, b1, W2)
    part2 = _sc_scatter(xs2, src_r, dst_r, zeros_rows)
    return _tc_head(part2, xs2, dis, b2, Wl1, bl1, Wl2, bl2)


# speedup vs baseline: 24.4560x; 1.8036x over previous
import functools

import jax
import jax.numpy as jnp
from jax import lax
from jax.experimental import pallas as pl
from jax.experimental.pallas import tpu as pltpu
from jax.experimental.pallas import tpu_sc as plsc

N = 10000
E = 320000
D = 128
H = 128
C = 64

NC = 2
NS = 16
NW = NC * NS
EW = E // NW
K = 80
NCHUNK = EW // K
NP = 10240
RPW = NP // NS
RCH = K
NRCH = RPW // RCH

BN = 1024
GRID = (N + BN - 1) // BN

_mesh = plsc.VectorSubcoreMesh(
    core_axis_name="c", subcore_axis_name="s", num_cores=NC, num_subcores=NS)

_HIGH = jax.lax.Precision.HIGHEST

_SC_PARAMS = pltpu.CompilerParams(needs_layout_passes=False)


@functools.partial(
    pl.kernel,
    out_type=jax.ShapeDtypeStruct((NW, N), jnp.float32),
    mesh=_mesh,
    compiler_params=_SC_PARAMS,
    scratch_types=[
        pltpu.VMEM((EW,), jnp.int32),
        pltpu.VMEM((N,), jnp.float32),
    ],
)
def _sc_degree(dst_hbm, zeros_hbm, out_hbm, dst_v, hist_v):
    c = lax.axis_index("c")
    s = lax.axis_index("s")
    wid = c * NS + s
    pltpu.sync_copy(dst_hbm.at[wid], dst_v)
    pltpu.sync_copy(zeros_hbm, hist_v)
    ones = jnp.full((16,), 1.0, jnp.float32)

    def body(i, _):
        idx = dst_v[pl.ds(i * 16, 16)]
        plsc.addupdate_scatter(hist_v, [idx], ones)
        return _

    lax.fori_loop(0, EW // 16, body, 0, unroll=4)
    pltpu.sync_copy(hist_v, out_hbm.at[wid])


@functools.partial(
    pl.kernel,
    out_type=jax.ShapeDtypeStruct((NC, NP, H), jnp.float32),
    mesh=_mesh,
    compiler_params=_SC_PARAMS,
    scratch_types=[
        pltpu.VMEM((NCHUNK, K), jnp.int32),
        pltpu.VMEM((K,), jnp.int32),
        pltpu.VMEM((K,), jnp.int32),
        pltpu.VMEM((K, H), jnp.float32),
        pltpu.VMEM((K, H), jnp.float32),
        pltpu.VMEM_SHARED((NP, H), jnp.float32),
        pltpu.SemaphoreType.DMA,
        pltpu.SemaphoreType.DMA,
        pltpu.SemaphoreType.DMA,
        pltpu.SemaphoreType.DMA,
    ],
)
def _sc_scatter(xs_hbm, src_hbm, dst_hbm, zrows_hbm, out_hbm,
                sidx_v, didx0_v, didx1_v, rows0_v, rows1_v, acc_sh,
                gsem0, gsem1, dsem0, dsem1):
    c = lax.axis_index("c")
    s = lax.axis_index("s")
    wid = c * NS + s

    pltpu.async_copy(src_hbm.at[wid], sidx_v, gsem0)
    pltpu.sync_copy(zrows_hbm, rows0_v)
    for z in range(NRCH):
        pltpu.sync_copy(rows0_v, acc_sh.at[pl.ds(s * RPW + z * RCH, RCH)])
    pltpu.make_async_copy(src_hbm.at[wid], sidx_v, gsem0).wait()
    plsc.subcore_barrier()

    pltpu.async_copy(dst_hbm.at[wid, 0], didx0_v, dsem0)
    pltpu.async_copy(dst_hbm.at[wid, 1], didx1_v, dsem1)
    pltpu.async_copy(xs_hbm.at[sidx_v.at[0]], rows0_v, gsem0)

    def step(j, cur_rows, cur_gsem, cur_didx, cur_dsem, nxt_rows, nxt_gsem):
        pltpu.make_async_copy(
            xs_hbm.at[sidx_v.at[j]], cur_rows, cur_gsem).wait()

        @pl.when(j + 1 < NCHUNK)
        def _():
            pltpu.async_copy(xs_hbm.at[sidx_v.at[j + 1]], nxt_rows, nxt_gsem)

        pltpu.make_async_copy(dst_hbm.at[wid, j], cur_didx, cur_dsem).wait()
        pltpu.sync_copy(cur_rows, acc_sh.at[cur_didx], add=True)

        @pl.when(j + 2 < NCHUNK)
        def _():
            pltpu.async_copy(dst_hbm.at[wid, j + 2], cur_didx, cur_dsem)

    def body(j, carry):
        @pl.when(j % 2 == 0)
        def _():
            step(j, rows0_v, gsem0, didx0_v, dsem0, rows1_v, gsem1)

        @pl.when(j % 2 == 1)
        def _():
            step(j, rows1_v, gsem1, didx1_v, dsem1, rows0_v, gsem0)

        return carry

    lax.fori_loop(0, NCHUNK, body, 0)
    plsc.subcore_barrier()

    for z in range(NRCH):
        base = s * RPW + z * RCH
        pltpu.sync_copy(acc_sh.at[pl.ds(base, RCH)], rows0_v)
        pltpu.sync_copy(rows0_v, out_hbm.at[c, pl.ds(base, RCH)])


def _tc_pre_body(deg_ref, x_ref, w_ref, xs_ref, dis_ref):
    deg = jnp.sum(deg_ref[...], axis=0) + 1.0
    dis = lax.rsqrt(deg)
    xw = jnp.dot(x_ref[...], w_ref[...],
                 preferred_element_type=jnp.float32, precision=_HIGH)
    xs_ref[...] = xw * dis[:, None]
    dis_ref[...] = dis


def _tc_mid_body(p_ref, xs_ref, dis_ref, b_ref, w_ref, o_ref):
    dis = dis_ref[...]
    h = (p_ref[0] + p_ref[1] + xs_ref[...]) * dis[:, None] + b_ref[...]
    h = jnp.maximum(h, 0.0)
    o_ref[...] = jnp.dot(h, w_ref[...],
                         preferred_element_type=jnp.float32,
                         precision=_HIGH) * dis[:, None]


def _tc_head_body(p_ref, xs_ref, dis_ref, b_ref, w1_ref, b1_ref, w2_ref,
                  b2_ref, o_ref):
    dis = dis_ref[...]
    h = (p_ref[0] + p_ref[1] + xs_ref[...]) * dis[:, None] + b_ref[...]
    h = jnp.maximum(h, 0.0)
    h = jnp.dot(h, w1_ref[...], preferred_element_type=jnp.float32,
                precision=_HIGH) + b1_ref[...]
    h = jnp.maximum(h, 0.0)
    lg = jnp.dot(h, w2_ref[...], preferred_element_type=jnp.float32,
                 precision=_HIGH) + b2_ref[...]
    m = jnp.max(lg, axis=-1, keepdims=True)
    lg = lg - m
    o_ref[...] = lg - jnp.log(jnp.sum(jnp.exp(lg), axis=-1, keepdims=True))


def _rows(i):
    return (i, 0)


def _full2(i):
    return (0, 0)


def _full1(i):
    return (0,)


_tc_pre = pl.pallas_call(
    _tc_pre_body,
    grid=(GRID,),
    in_specs=[
        pl.BlockSpec((NW, BN), lambda i: (0, i)),
        pl.BlockSpec((BN, D), _rows),
        pl.BlockSpec((D, H), _full2),
    ],
    out_specs=[
        pl.BlockSpec((BN, H), _rows),
        pl.BlockSpec((BN,), lambda i: (i,)),
    ],
    out_shape=[
        jax.ShapeDtypeStruct((N, H), jnp.float32),
        jax.ShapeDtypeStruct((N,), jnp.float32),
    ],
)

_tc_mid = pl.pallas_call(
    _tc_mid_body,
    grid=(GRID,),
    in_specs=[
        pl.BlockSpec((NC, BN, H), lambda i: (0, i, 0)),
        pl.BlockSpec((BN, H), _rows),
        pl.BlockSpec((BN,), lambda i: (i,)),
        pl.BlockSpec((H,), _full1),
        pl.BlockSpec((H, H), _full2),
    ],
    out_specs=pl.BlockSpec((BN, H), _rows),
    out_shape=jax.ShapeDtypeStruct((N, H), jnp.float32),
)

_tc_head = pl.pallas_call(
    _tc_head_body,
    grid=(GRID,),
    in_specs=[
        pl.BlockSpec((NC, BN, H), lambda i: (0, i, 0)),
        pl.BlockSpec((BN, H), _rows),
        pl.BlockSpec((BN,), lambda i: (i,)),
        pl.BlockSpec((H,), _full1),
        pl.BlockSpec((H, H), _full2),
        pl.BlockSpec((H,), _full1),
        pl.BlockSpec((H, C), _full2),
        pl.BlockSpec((C,), _full1),
    ],
    out_specs=pl.BlockSpec((BN, C), _rows),
    out_shape=jax.ShapeDtypeStruct((N, C), jnp.float32),
)


def kernel(x, edge_index, W1, b1, W2, b2, Wl1, bl1, Wl2, bl2):
    src_r = edge_index[0].reshape(NW, NCHUNK, K)
    dst_r = edge_index[1].reshape(NW, NCHUNK, K)
    dst_f = edge_index[1].reshape(NW, EW)
    zeros_n = jnp.zeros((N,), jnp.float32)
    zeros_rows = jnp.zeros((RCH, H), jnp.float32)

    deg_part = _sc_degree(dst_f, zeros_n)
    xs1, dis = _tc_pre(deg_part, x, W1)
    part1 = _sc_scatter(xs1, src_r, dst_r, zeros_rows)
    xs2 = _tc_mid(part1, xs1, dis, b1, W2)
    part2 = _sc_scatter(xs2, src_r, dst_r, zeros_rows)
    return _tc_head(part2, xs2, dis, b2, Wl1, bl1, Wl2, bl2)


# speedup vs baseline: 24.6675x; 1.0087x over previous
import functools

import jax
import jax.numpy as jnp
from jax import lax
from jax.experimental import pallas as pl
from jax.experimental.pallas import tpu as pltpu
from jax.experimental.pallas import tpu_sc as plsc

N = 10000
E = 320000
D = 128
H = 128
C = 64

NC = 2
NS = 16
NW = NC * NS
EW = E // NW
K = 80
NCHUNK = EW // K
NP = 10240
RPW = NP // NS
RCH = K
NRCH = RPW // RCH

BN = 1024
GRID = (N + BN - 1) // BN

_mesh = plsc.VectorSubcoreMesh(
    core_axis_name="c", subcore_axis_name="s", num_cores=NC, num_subcores=NS)

_HIGH = jax.lax.Precision.HIGHEST

_SC_PARAMS = pltpu.CompilerParams(needs_layout_passes=False)


@functools.partial(
    pl.kernel,
    out_type=jax.ShapeDtypeStruct((NW, N), jnp.float32),
    mesh=_mesh,
    compiler_params=_SC_PARAMS,
    scratch_types=[
        pltpu.VMEM((EW,), jnp.int32),
        pltpu.VMEM((N,), jnp.float32),
    ],
)
def _sc_degree(dst_hbm, zeros_hbm, out_hbm, dst_v, hist_v):
    c = lax.axis_index("c")
    s = lax.axis_index("s")
    wid = c * NS + s
    pltpu.sync_copy(dst_hbm.at[wid], dst_v)
    pltpu.sync_copy(zeros_hbm, hist_v)
    ones = jnp.full((16,), 1.0, jnp.float32)

    def body(i, _):
        idx = dst_v[pl.ds(i * 16, 16)]
        plsc.addupdate_scatter(hist_v, [idx], ones)
        return _

    lax.fori_loop(0, EW // 16, body, 0, unroll=4)
    pltpu.sync_copy(hist_v, out_hbm.at[wid])


@functools.partial(
    pl.kernel,
    out_type=jax.ShapeDtypeStruct((NC, NP, H), jnp.float32),
    mesh=_mesh,
    compiler_params=_SC_PARAMS,
    scratch_types=[
        pltpu.VMEM((K,), jnp.int32),
        pltpu.VMEM((K,), jnp.int32),
        pltpu.VMEM((K,), jnp.int32),
        pltpu.VMEM((K,), jnp.int32),
        pltpu.VMEM((K, H), jnp.float32),
        pltpu.VMEM((K, H), jnp.float32),
        pltpu.VMEM_SHARED((NP, H), jnp.float32),
        pltpu.SemaphoreType.DMA,
        pltpu.SemaphoreType.DMA,
        pltpu.SemaphoreType.DMA,
        pltpu.SemaphoreType.DMA,
        pltpu.SemaphoreType.DMA,
        pltpu.SemaphoreType.DMA,
        pltpu.SemaphoreType.DMA,
        pltpu.SemaphoreType.DMA,
    ],
)
def _sc_scatter(xs_hbm, src_hbm, dst_hbm, zrows_hbm, out_hbm,
                sidx0_v, sidx1_v, didx0_v, didx1_v, rows0_v, rows1_v, acc_sh,
                gsem0, gsem1, dsem0, dsem1, sisem0, sisem1, ssem0, ssem1):
    c = lax.axis_index("c")
    s = lax.axis_index("s")
    wid = c * NS + s
    sidx = (sidx0_v, sidx1_v)
    didx = (didx0_v, didx1_v)
    rows = (rows0_v, rows1_v)
    gsem = (gsem0, gsem1)
    dsem = (dsem0, dsem1)
    sisem = (sisem0, sisem1)
    ssem = (ssem0, ssem1)

    pltpu.sync_copy(zrows_hbm, rows0_v)
    for z in range(NRCH):
        pltpu.async_copy(
            rows0_v, acc_sh.at[pl.ds(s * RPW + z * RCH, RCH)], gsem0)
    for z in range(NRCH):
        pltpu.make_async_copy(
            rows0_v, acc_sh.at[pl.ds(s * RPW, RCH)], gsem0).wait()
    plsc.subcore_barrier()

    pltpu.async_copy(src_hbm.at[wid, 0], sidx0_v, sisem0)
    pltpu.async_copy(src_hbm.at[wid, 1], sidx1_v, sisem1)
    pltpu.async_copy(dst_hbm.at[wid, 0], didx0_v, dsem0)
    pltpu.async_copy(dst_hbm.at[wid, 1], didx1_v, dsem1)
    pltpu.make_async_copy(src_hbm.at[wid, 0], sidx0_v, sisem0).wait()
    pltpu.async_copy(xs_hbm.at[sidx0_v], rows0_v, gsem0)

    def step(j, p):
        q = 1 - p
        pltpu.make_async_copy(xs_hbm.at[sidx[p]], rows[p], gsem[p]).wait()

        @pl.when(j >= 1)
        def _():
            pltpu.make_async_copy(
                rows[q], acc_sh.at[didx[q]], ssem[q]).wait()

            @pl.when(j + 1 < NCHUNK)
            def _():
                pltpu.async_copy(dst_hbm.at[wid, j + 1], didx[q], dsem[q])

        @pl.when(j + 1 < NCHUNK)
        def _():
            pltpu.make_async_copy(
                src_hbm.at[wid, j + 1], sidx[q], sisem[q]).wait()
            pltpu.async_copy(xs_hbm.at[sidx[q]], rows[q], gsem[q])

        @pl.when(j + 2 < NCHUNK)
        def _():
            pltpu.async_copy(src_hbm.at[wid, j + 2], sidx[p], sisem[p])

        pltpu.make_async_copy(dst_hbm.at[wid, j], didx[p], dsem[p]).wait()
        pltpu.async_copy(rows[p], acc_sh.at[didx[p]], ssem[p], add=True)

    def body(j, carry):
        @pl.when(j % 2 == 0)
        def _():
            step(j, 0)

        @pl.when(j % 2 == 1)
        def _():
            step(j, 1)

        return carry

    lax.fori_loop(0, NCHUNK, body, 0)
    pltpu.make_async_copy(rows0_v, acc_sh.at[didx0_v], ssem0).wait()
    plsc.subcore_barrier()

    for z in range(NRCH):
        p = z & 1
        base = s * RPW + z * RCH
        if z >= 2:
            pltpu.make_async_copy(
                rows[p], out_hbm.at[c, pl.ds(base, RCH)], gsem[p]).wait()
        pltpu.sync_copy(acc_sh.at[pl.ds(base, RCH)], rows[p])
        pltpu.async_copy(rows[p], out_hbm.at[c, pl.ds(base, RCH)], gsem[p])
    pltpu.make_async_copy(
        rows0_v, out_hbm.at[c, pl.ds(s * RPW, RCH)], gsem0).wait()
    pltpu.make_async_copy(
        rows1_v, out_hbm.at[c, pl.ds(s * RPW, RCH)], gsem1).wait()


def _tc_pre_body(deg_ref, x_ref, w_ref, xs_ref, dis_ref):
    deg = jnp.sum(deg_ref[...], axis=0) + 1.0
    dis = lax.rsqrt(deg)
    xw = jnp.dot(x_ref[...], w_ref[...],
                 preferred_element_type=jnp.float32, precision=_HIGH)
    xs_ref[...] = xw * dis[:, None]
    dis_ref[...] = dis


def _tc_mid_body(p_ref, xs_ref, dis_ref, b_ref, w_ref, o_ref):
    dis = dis_ref[...]
    h = (p_ref[0] + p_ref[1] + xs_ref[...]) * dis[:, None] + b_ref[...]
    h = jnp.maximum(h, 0.0)
    o_ref[...] = jnp.dot(h, w_ref[...],
                         preferred_element_type=jnp.float32,
                         precision=_HIGH) * dis[:, None]


def _tc_head_body(p_ref, xs_ref, dis_ref, b_ref, w1_ref, b1_ref, w2_ref,
                  b2_ref, o_ref):
    dis = dis_ref[...]
    h = (p_ref[0] + p_ref[1] + xs_ref[...]) * dis[:, None] + b_ref[...]
    h = jnp.maximum(h, 0.0)
    h = jnp.dot(h, w1_ref[...], preferred_element_type=jnp.float32,
                precision=_HIGH) + b1_ref[...]
    h = jnp.maximum(h, 0.0)
    lg = jnp.dot(h, w2_ref[...], preferred_element_type=jnp.float32,
                 precision=_HIGH) + b2_ref[...]
    m = jnp.max(lg, axis=-1, keepdims=True)
    lg = lg - m
    o_ref[...] = lg - jnp.log(jnp.sum(jnp.exp(lg), axis=-1, keepdims=True))


def _rows(i):
    return (i, 0)


def _full2(i):
    return (0, 0)


def _full1(i):
    return (0,)


_tc_pre = pl.pallas_call(
    _tc_pre_body,
    grid=(GRID,),
    in_specs=[
        pl.BlockSpec((NW, BN), lambda i: (0, i)),
        pl.BlockSpec((BN, D), _rows),
        pl.BlockSpec((D, H), _full2),
    ],
    out_specs=[
        pl.BlockSpec((BN, H), _rows),
        pl.BlockSpec((BN,), lambda i: (i,)),
    ],
    out_shape=[
        jax.ShapeDtypeStruct((N, H), jnp.float32),
        jax.ShapeDtypeStruct((N,), jnp.float32),
    ],
)

_tc_mid = pl.pallas_call(
    _tc_mid_body,
    grid=(GRID,),
    in_specs=[
        pl.BlockSpec((NC, BN, H), lambda i: (0, i, 0)),
        pl.BlockSpec((BN, H), _rows),
        pl.BlockSpec((BN,), lambda i: (i,)),
        pl.BlockSpec((H,), _full1),
        pl.BlockSpec((H, H), _full2),
    ],
    out_specs=pl.BlockSpec((BN, H), _rows),
    out_shape=jax.ShapeDtypeStruct((N, H), jnp.float32),
)

_tc_head = pl.pallas_call(
    _tc_head_body,
    grid=(GRID,),
    in_specs=[
        pl.BlockSpec((NC, BN, H), lambda i: (0, i, 0)),
        pl.BlockSpec((BN, H), _rows),
        pl.BlockSpec((BN,), lambda i: (i,)),
        pl.BlockSpec((H,), _full1),
        pl.BlockSpec((H, H), _full2),
        pl.BlockSpec((H,), _full1),
        pl.BlockSpec((H, C), _full2),
        pl.BlockSpec((C,), _full1),
    ],
    out_specs=pl.BlockSpec((BN, C), _rows),
    out_shape=jax.ShapeDtypeStruct((N, C), jnp.float32),
)


def kernel(x, edge_index, W1, b1, W2, b2, Wl1, bl1, Wl2, bl2):
    src_r = edge_index[0].reshape(NW, NCHUNK, K)
    dst_r = edge_index[1].reshape(NW, NCHUNK, K)
    dst_f = edge_index[1].reshape(NW, EW)
    zeros_n = jnp.zeros((N,), jnp.float32)
    zeros_rows = jnp.zeros((RCH, H), jnp.float32)

    deg_part = _sc_degree(dst_f, zeros_n)
    xs1, dis = _tc_pre(deg_part, x, W1)
    part1 = _sc_scatter(xs1, src_r, dst_r, zeros_rows)
    xs2 = _tc_mid(part1, xs1, dis, b1, W2)
    part2 = _sc_scatter(xs2, src_r, dst_r, zeros_rows)
    return _tc_head(part2, xs2, dis, b2, Wl1, bl1, Wl2, bl2)


# speedup vs baseline: 30.9692x; 1.2555x over previous
import functools

import jax
import jax.numpy as jnp
from jax import lax
from jax.experimental import pallas as pl
from jax.experimental.pallas import tpu as pltpu
from jax.experimental.pallas import tpu_sc as plsc

N = 10000
E = 320000
D = 128
H = 128
C = 64

NC = 2
NS = 16
NW = NC * NS
EW = E // NW
K = 80
NCHUNK = EW // K
NP = 10240
RPW = NP // NS
RCH = K
NRCH = RPW // RCH

BN = 1024
GRID = (N + BN - 1) // BN

_mesh = plsc.VectorSubcoreMesh(
    core_axis_name="c", subcore_axis_name="s", num_cores=NC, num_subcores=NS)

_HIGH = jax.lax.Precision.HIGHEST

_SC_PARAMS = pltpu.CompilerParams(needs_layout_passes=False)


@functools.partial(
    pl.kernel,
    out_type=jax.ShapeDtypeStruct((NW, N), jnp.float32),
    mesh=_mesh,
    compiler_params=_SC_PARAMS,
    scratch_types=[
        pltpu.VMEM((EW,), jnp.int32),
        pltpu.VMEM((N,), jnp.float32),
    ],
)
def _sc_degree(dst_hbm, zeros_hbm, out_hbm, dst_v, hist_v):
    c = lax.axis_index("c")
    s = lax.axis_index("s")
    wid = c * NS + s
    pltpu.sync_copy(dst_hbm.at[wid], dst_v)
    pltpu.sync_copy(zeros_hbm, hist_v)
    ones = jnp.full((16,), 1.0, jnp.float32)

    def body(i, _):
        idx = dst_v[pl.ds(i * 16, 16)]
        plsc.addupdate_scatter(hist_v, [idx], ones)
        return _

    lax.fori_loop(0, EW // 16, body, 0, unroll=4)
    pltpu.sync_copy(hist_v, out_hbm.at[wid])


@functools.partial(
    pl.kernel,
    out_type=jax.ShapeDtypeStruct((NC, NP, H), jnp.float32),
    mesh=_mesh,
    compiler_params=_SC_PARAMS,
    scratch_types=(
        [pltpu.VMEM((K,), jnp.int32)] * 4 +
        [pltpu.VMEM((K,), jnp.int32)] * 4 +
        [pltpu.VMEM((K, H), jnp.float32)] * 4 +
        [pltpu.VMEM_SHARED((NP, H), jnp.float32)] +
        [pltpu.SemaphoreType.DMA] * 16
    ),
)
def _sc_scatter(xs_hbm, src_hbm, dst_hbm, zrows_hbm, out_hbm,
                si0, si1, si2, si3, di0, di1, di2, di3, r0, r1, r2, r3,
                acc_sh, *sems):
    c = lax.axis_index("c")
    s = lax.axis_index("s")
    wid = c * NS + s
    sidx = (si0, si1, si2, si3)
    didx = (di0, di1, di2, di3)
    rows = (r0, r1, r2, r3)
    gsem = sems[0:4]
    dsem = sems[4:8]
    sisem = sems[8:12]
    ssem = sems[12:16]

    pltpu.sync_copy(zrows_hbm, r0)
    for z in range(NRCH):
        pltpu.async_copy(
            r0, acc_sh.at[pl.ds(s * RPW + z * RCH, RCH)], gsem[0])
    for z in range(NRCH):
        pltpu.make_async_copy(
            r0, acc_sh.at[pl.ds(s * RPW, RCH)], gsem[0]).wait()
    plsc.subcore_barrier()

    for jj in range(4):
        pltpu.async_copy(src_hbm.at[wid, jj], sidx[jj], sisem[jj])
    for jj in range(2):
        pltpu.async_copy(dst_hbm.at[wid, jj], didx[jj], dsem[jj])
        pltpu.make_async_copy(src_hbm.at[wid, jj], sidx[jj], sisem[jj]).wait()
        pltpu.async_copy(xs_hbm.at[sidx[jj]], rows[jj], gsem[jj])

    def step(j, p):
        p2 = (p + 2) % 4
        pltpu.make_async_copy(xs_hbm.at[sidx[p]], rows[p], gsem[p]).wait()

        @pl.when(j >= 2)
        def _():
            pltpu.make_async_copy(
                rows[p2], acc_sh.at[didx[p2]], ssem[p2]).wait()

        pltpu.make_async_copy(dst_hbm.at[wid, j], didx[p], dsem[p]).wait()
        pltpu.async_copy(rows[p], acc_sh.at[didx[p]], ssem[p], add=True)

        @pl.when(j + 4 < NCHUNK)
        def _():
            pltpu.async_copy(src_hbm.at[wid, j + 4], sidx[p], sisem[p])

        @pl.when(j + 2 < NCHUNK)
        def _():
            pltpu.async_copy(dst_hbm.at[wid, j + 2], didx[p2], dsem[p2])
            pltpu.make_async_copy(
                src_hbm.at[wid, j + 2], sidx[p2], sisem[p2]).wait()
            pltpu.async_copy(xs_hbm.at[sidx[p2]], rows[p2], gsem[p2])

    def body(j, carry):
        for pp in range(4):
            @pl.when(j % 4 == pp)
            def _(pp=pp):
                step(j, pp)

        return carry

    lax.fori_loop(0, NCHUNK, body, 0)
    for jj in (NCHUNK - 2, NCHUNK - 1):
        pltpu.make_async_copy(
            rows[jj % 4], acc_sh.at[didx[jj % 4]], ssem[jj % 4]).wait()
    plsc.subcore_barrier()

    for z in range(NRCH):
        p = z & 1
        base = s * RPW + z * RCH
        if z >= 2:
            pltpu.make_async_copy(
                rows[p], out_hbm.at[c, pl.ds(base, RCH)], gsem[p]).wait()
        pltpu.sync_copy(acc_sh.at[pl.ds(base, RCH)], rows[p])
        pltpu.async_copy(rows[p], out_hbm.at[c, pl.ds(base, RCH)], gsem[p])
    pltpu.make_async_copy(
        r0, out_hbm.at[c, pl.ds(s * RPW, RCH)], gsem[0]).wait()
    pltpu.make_async_copy(
        r1, out_hbm.at[c, pl.ds(s * RPW, RCH)], gsem[1]).wait()


def _tc_pre_body(deg_ref, x_ref, w_ref, xs_ref, dis_ref):
    deg = jnp.sum(deg_ref[...], axis=0) + 1.0
    dis = lax.rsqrt(deg)
    xw = jnp.dot(x_ref[...], w_ref[...],
                 preferred_element_type=jnp.float32, precision=_HIGH)
    xs_ref[...] = xw * dis[:, None]
    dis_ref[...] = dis


def _tc_mid_body(p_ref, xs_ref, dis_ref, b_ref, w_ref, o_ref):
    dis = dis_ref[...]
    h = (p_ref[0] + p_ref[1] + xs_ref[...]) * dis[:, None] + b_ref[...]
    h = jnp.maximum(h, 0.0)
    o_ref[...] = jnp.dot(h, w_ref[...],
                         preferred_element_type=jnp.float32,
                         precision=_HIGH) * dis[:, None]


def _tc_head_body(p_ref, xs_ref, dis_ref, b_ref, w1_ref, b1_ref, w2_ref,
                  b2_ref, o_ref):
    dis = dis_ref[...]
    h = (p_ref[0] + p_ref[1] + xs_ref[...]) * dis[:, None] + b_ref[...]
    h = jnp.maximum(h, 0.0)
    h = jnp.dot(h, w1_ref[...], preferred_element_type=jnp.float32,
                precision=_HIGH) + b1_ref[...]
    h = jnp.maximum(h, 0.0)
    lg = jnp.dot(h, w2_ref[...], preferred_element_type=jnp.float32,
                 precision=_HIGH) + b2_ref[...]
    m = jnp.max(lg, axis=-1, keepdims=True)
    lg = lg - m
    o_ref[...] = lg - jnp.log(jnp.sum(jnp.exp(lg), axis=-1, keepdims=True))


def _rows(i):
    return (i, 0)


def _full2(i):
    return (0, 0)


def _full1(i):
    return (0,)


_tc_pre = pl.pallas_call(
    _tc_pre_body,
    grid=(GRID,),
    in_specs=[
        pl.BlockSpec((NW, BN), lambda i: (0, i)),
        pl.BlockSpec((BN, D), _rows),
        pl.BlockSpec((D, H), _full2),
    ],
    out_specs=[
        pl.BlockSpec((BN, H), _rows),
        pl.BlockSpec((BN,), lambda i: (i,)),
    ],
    out_shape=[
        jax.ShapeDtypeStruct((N, H), jnp.float32),
        jax.ShapeDtypeStruct((N,), jnp.float32),
    ],
)

_tc_mid = pl.pallas_call(
    _tc_mid_body,
    grid=(GRID,),
    in_specs=[
        pl.BlockSpec((NC, BN, H), lambda i: (0, i, 0)),
        pl.BlockSpec((BN, H), _rows),
        pl.BlockSpec((BN,), lambda i: (i,)),
        pl.BlockSpec((H,), _full1),
        pl.BlockSpec((H, H), _full2),
    ],
    out_specs=pl.BlockSpec((BN, H), _rows),
    out_shape=jax.ShapeDtypeStruct((N, H), jnp.float32),
)

_tc_head = pl.pallas_call(
    _tc_head_body,
    grid=(GRID,),
    in_specs=[
        pl.BlockSpec((NC, BN, H), lambda i: (0, i, 0)),
        pl.BlockSpec((BN, H), _rows),
        pl.BlockSpec((BN,), lambda i: (i,)),
        pl.BlockSpec((H,), _full1),
        pl.BlockSpec((H, H), _full2),
        pl.BlockSpec((H,), _full1),
        pl.BlockSpec((H, C), _full2),
        pl.BlockSpec((C,), _full1),
    ],
    out_specs=pl.BlockSpec((BN, C), _rows),
    out_shape=jax.ShapeDtypeStruct((N, C), jnp.float32),
)


def kernel(x, edge_index, W1, b1, W2, b2, Wl1, bl1, Wl2, bl2):
    src_r = edge_index[0].reshape(NW, NCHUNK, K)
    dst_r = edge_index[1].reshape(NW, NCHUNK, K)
    dst_f = edge_index[1].reshape(NW, EW)
    zeros_n = jnp.zeros((N,), jnp.float32)
    zeros_rows = jnp.zeros((RCH, H), jnp.float32)

    deg_part = _sc_degree(dst_f, zeros_n)
    xs1, dis = _tc_pre(deg_part, x, W1)
    part1 = _sc_scatter(xs1, src_r, dst_r, zeros_rows)
    xs2 = _tc_mid(part1, xs1, dis, b1, W2)
    part2 = _sc_scatter(xs2, src_r, dst_r, zeros_rows)
    return _tc_head(part2, xs2, dis, b2, Wl1, bl1, Wl2, bl2)


# speedup vs baseline: 33.0507x; 1.0672x over previous
import functools

import jax
import jax.numpy as jnp
from jax import lax
from jax.experimental import pallas as pl
from jax.experimental.pallas import tpu as pltpu
from jax.experimental.pallas import tpu_sc as plsc

N = 10000
E = 320000
D = 128
H = 128
C = 64

NC = 2
NS = 16
NW = NC * NS
EW = E // NW
K = 40
NCHUNK = EW // K
NBUF = 6
GL = 3
NP = 10240
RPW = NP // NS
RCH = K
NRCH = RPW // RCH

BN = 2048
GRID = (NP + BN - 1) // BN

_mesh = plsc.VectorSubcoreMesh(
    core_axis_name="c", subcore_axis_name="s", num_cores=NC, num_subcores=NS)

_HIGH = jax.lax.Precision.HIGHEST

_SC_PARAMS = pltpu.CompilerParams(needs_layout_passes=False)


@functools.partial(
    pl.kernel,
    out_type=jax.ShapeDtypeStruct((NW, N), jnp.float32),
    mesh=_mesh,
    compiler_params=_SC_PARAMS,
    scratch_types=[
        pltpu.VMEM((EW,), jnp.int32),
        pltpu.VMEM((N,), jnp.float32),
    ],
)
def _sc_degree(dst_hbm, zeros_hbm, out_hbm, dst_v, hist_v):
    c = lax.axis_index("c")
    s = lax.axis_index("s")
    wid = c * NS + s
    pltpu.sync_copy(dst_hbm.at[wid], dst_v)
    pltpu.sync_copy(zeros_hbm, hist_v)
    ones = jnp.full((16,), 1.0, jnp.float32)

    def body(i, _):
        idx = dst_v[pl.ds(i * 16, 16)]
        plsc.addupdate_scatter(hist_v, [idx], ones)
        return _

    lax.fori_loop(0, EW // 16, body, 0, unroll=4)
    pltpu.sync_copy(hist_v, out_hbm.at[wid])


@functools.partial(
    pl.kernel,
    out_type=jax.ShapeDtypeStruct((NC, NP, H), jnp.float32),
    mesh=_mesh,
    compiler_params=_SC_PARAMS,
    scratch_types=(
        [pltpu.VMEM((K,), jnp.int32)] * NBUF +
        [pltpu.VMEM((K,), jnp.int32)] * NBUF +
        [pltpu.VMEM((K, H), jnp.float32)] * NBUF +
        [pltpu.VMEM_SHARED((NP, H), jnp.float32)] +
        [pltpu.SemaphoreType.DMA] * (4 * NBUF)
    ),
)
def _sc_scatter(xs_hbm, src_hbm, dst_hbm, zrows_hbm, out_hbm, *scr):
    c = lax.axis_index("c")
    s = lax.axis_index("s")
    wid = c * NS + s
    sidx = scr[0:NBUF]
    didx = scr[NBUF:2 * NBUF]
    rows = scr[2 * NBUF:3 * NBUF]
    acc_sh = scr[3 * NBUF]
    sems = scr[3 * NBUF + 1:]
    gsem = sems[0:NBUF]
    dsem = sems[NBUF:2 * NBUF]
    sisem = sems[2 * NBUF:3 * NBUF]
    ssem = sems[3 * NBUF:4 * NBUF]
    r0, r1 = rows[0], rows[1]

    pltpu.sync_copy(zrows_hbm, r0)
    for z in range(NRCH):
        pltpu.async_copy(
            r0, acc_sh.at[pl.ds(s * RPW + z * RCH, RCH)], gsem[0])
    for z in range(NRCH):
        pltpu.make_async_copy(
            r0, acc_sh.at[pl.ds(s * RPW, RCH)], gsem[0]).wait()
    plsc.subcore_barrier()

    for jj in range(NBUF):
        pltpu.async_copy(src_hbm.at[wid, jj], sidx[jj], sisem[jj])
    for jj in range(GL):
        pltpu.async_copy(dst_hbm.at[wid, jj], didx[jj], dsem[jj])
        pltpu.make_async_copy(src_hbm.at[wid, jj], sidx[jj], sisem[jj]).wait()
        pltpu.async_copy(xs_hbm.at[sidx[jj]], rows[jj], gsem[jj])

    def step(j, p):
        pg = (p + GL) % NBUF
        pltpu.make_async_copy(xs_hbm.at[sidx[p]], rows[p], gsem[p]).wait()

        @pl.when(j >= NBUF - GL)
        def _():
            pltpu.make_async_copy(
                rows[pg], acc_sh.at[didx[pg]], ssem[pg]).wait()

        pltpu.make_async_copy(dst_hbm.at[wid, j], didx[p], dsem[p]).wait()
        pltpu.async_copy(rows[p], acc_sh.at[didx[p]], ssem[p], add=True)

        @pl.when(j + NBUF < NCHUNK)
        def _():
            pltpu.async_copy(src_hbm.at[wid, j + NBUF], sidx[p], sisem[p])

        @pl.when(j + GL < NCHUNK)
        def _():
            pltpu.async_copy(dst_hbm.at[wid, j + GL], didx[pg], dsem[pg])
            pltpu.make_async_copy(
                src_hbm.at[wid, j + GL], sidx[pg], sisem[pg]).wait()
            pltpu.async_copy(xs_hbm.at[sidx[pg]], rows[pg], gsem[pg])

    def body(j, carry):
        for pp in range(NBUF):
            @pl.when(j % NBUF == pp)
            def _(pp=pp):
                step(j, pp)

        return carry

    lax.fori_loop(0, NCHUNK, body, 0)
    for jj in range(NCHUNK - (NBUF - GL), NCHUNK):
        pltpu.make_async_copy(
            rows[jj % NBUF], acc_sh.at[didx[jj % NBUF]], ssem[jj % NBUF]).wait()
    plsc.subcore_barrier()

    for z in range(NRCH):
        p = z & 1
        base = s * RPW + z * RCH
        if z >= 2:
            pltpu.make_async_copy(
                rows[p], out_hbm.at[c, pl.ds(base, RCH)], gsem[p]).wait()
        pltpu.sync_copy(acc_sh.at[pl.ds(base, RCH)], rows[p])
        pltpu.async_copy(rows[p], out_hbm.at[c, pl.ds(base, RCH)], gsem[p])
    pltpu.make_async_copy(
        r0, out_hbm.at[c, pl.ds(s * RPW, RCH)], gsem[0]).wait()
    pltpu.make_async_copy(
        r1, out_hbm.at[c, pl.ds(s * RPW, RCH)], gsem[1]).wait()


def _tc_pre_body(deg_ref, x_ref, w_ref, xs_ref, dis_ref):
    deg = jnp.sum(deg_ref[...], axis=0) + 1.0
    dis = lax.rsqrt(deg)
    xw = jnp.dot(x_ref[...], w_ref[...],
                 preferred_element_type=jnp.float32, precision=_HIGH)
    xs_ref[...] = xw * dis[:, None]
    dis_ref[...] = dis


def _tc_mid_body(p_ref, xs_ref, dis_ref, b_ref, w_ref, o_ref):
    dis = dis_ref[...]
    h = (p_ref[0] + p_ref[1] + xs_ref[...]) * dis[:, None] + b_ref[...]
    h = jnp.maximum(h, 0.0)
    o_ref[...] = jnp.dot(h, w_ref[...],
                         preferred_element_type=jnp.float32,
                         precision=_HIGH) * dis[:, None]


def _tc_head_body(p_ref, xs_ref, dis_ref, b_ref, w1_ref, b1_ref, w2_ref,
                  b2_ref, o_ref):
    dis = dis_ref[...]
    h = (p_ref[0] + p_ref[1] + xs_ref[...]) * dis[:, None] + b_ref[...]
    h = jnp.maximum(h, 0.0)
    h = jnp.dot(h, w1_ref[...], preferred_element_type=jnp.float32,
                precision=_HIGH) + b1_ref[...]
    h = jnp.maximum(h, 0.0)
    lg = jnp.dot(h, w2_ref[...], preferred_element_type=jnp.float32,
                 precision=_HIGH) + b2_ref[...]
    m = jnp.max(lg, axis=-1, keepdims=True)
    lg = lg - m
    o_ref[...] = lg - jnp.log(jnp.sum(jnp.exp(lg), axis=-1, keepdims=True))


def _rows(i):
    return (i, 0)


def _full2(i):
    return (0, 0)


def _full1(i):
    return (0,)


_tc_pre = pl.pallas_call(
    _tc_pre_body,
    grid=(GRID,),
    in_specs=[
        pl.BlockSpec((NW, BN), lambda i: (0, i)),
        pl.BlockSpec((BN, D), _rows),
        pl.BlockSpec((D, H), _full2),
    ],
    out_specs=[
        pl.BlockSpec((BN, H), _rows),
        pl.BlockSpec((BN,), lambda i: (i,)),
    ],
    out_shape=[
        jax.ShapeDtypeStruct((N, H), jnp.float32),
        jax.ShapeDtypeStruct((N,), jnp.float32),
    ],
)

_tc_mid = pl.pallas_call(
    _tc_mid_body,
    grid=(GRID,),
    in_specs=[
        pl.BlockSpec((NC, BN, H), lambda i: (0, i, 0)),
        pl.BlockSpec((BN, H), _rows),
        pl.BlockSpec((BN,), lambda i: (i,)),
        pl.BlockSpec((H,), _full1),
        pl.BlockSpec((H, H), _full2),
    ],
    out_specs=pl.BlockSpec((BN, H), _rows),
    out_shape=jax.ShapeDtypeStruct((N, H), jnp.float32),
)

_tc_head = pl.pallas_call(
    _tc_head_body,
    grid=(GRID,),
    in_specs=[
        pl.BlockSpec((NC, BN, H), lambda i: (0, i, 0)),
        pl.BlockSpec((BN, H), _rows),
        pl.BlockSpec((BN,), lambda i: (i,)),
        pl.BlockSpec((H,), _full1),
        pl.BlockSpec((H, H), _full2),
        pl.BlockSpec((H,), _full1),
        pl.BlockSpec((H, C), _full2),
        pl.BlockSpec((C,), _full1),
    ],
    out_specs=pl.BlockSpec((BN, C), _rows),
    out_shape=jax.ShapeDtypeStruct((N, C), jnp.float32),
)


def kernel(x, edge_index, W1, b1, W2, b2, Wl1, bl1, Wl2, bl2):
    src_r = edge_index[0].reshape(NW, NCHUNK, K)
    dst_r = edge_index[1].reshape(NW, NCHUNK, K)
    dst_f = edge_index[1].reshape(NW, EW)
    zeros_n = jnp.zeros((N,), jnp.float32)
    zeros_rows = jnp.zeros((RCH, H), jnp.float32)

    deg_part = _sc_degree(dst_f, zeros_n)
    xs1, dis = _tc_pre(deg_part, x, W1)
    part1 = _sc_scatter(xs1, src_r, dst_r, zeros_rows)
    xs2 = _tc_mid(part1, xs1, dis, b1, W2)
    part2 = _sc_scatter(xs2, src_r, dst_r, zeros_rows)
    return _tc_head(part2, xs2, dis, b2, Wl1, bl1, Wl2, bl2)


# speedup vs baseline: 34.9569x; 1.0577x over previous
import functools

import jax
import jax.numpy as jnp
from jax import lax
from jax.experimental import pallas as pl
from jax.experimental.pallas import tpu as pltpu
from jax.experimental.pallas import tpu_sc as plsc

N = 10000
E = 320000
D = 128
H = 128
C = 64

NC = 2
NS = 16
NW = NC * NS
EW = E // NW
K = 40
NCHUNK = EW // K
NBUF = 6
GL = 4
NP = 10240
RPW = NP // NS
RCH = K
NRCH = RPW // RCH

BN = 2048
GRID = (NP + BN - 1) // BN

_mesh = plsc.VectorSubcoreMesh(
    core_axis_name="c", subcore_axis_name="s", num_cores=NC, num_subcores=NS)

_HIGH = jax.lax.Precision.HIGHEST

_SC_PARAMS = pltpu.CompilerParams(needs_layout_passes=False)


@functools.partial(
    pl.kernel,
    out_type=jax.ShapeDtypeStruct((NW, N), jnp.float32),
    mesh=_mesh,
    compiler_params=_SC_PARAMS,
    scratch_types=[
        pltpu.VMEM((EW,), jnp.int32),
        pltpu.VMEM((N,), jnp.float32),
    ],
)
def _sc_degree(dst_hbm, zeros_hbm, out_hbm, dst_v, hist_v):
    c = lax.axis_index("c")
    s = lax.axis_index("s")
    wid = c * NS + s
    pltpu.sync_copy(dst_hbm.at[wid], dst_v)
    pltpu.sync_copy(zeros_hbm, hist_v)
    ones = jnp.full((16,), 1.0, jnp.float32)

    def body(i, _):
        idx = dst_v[pl.ds(i * 16, 16)]
        plsc.addupdate_scatter(hist_v, [idx], ones)
        return _

    lax.fori_loop(0, EW // 16, body, 0, unroll=4)
    pltpu.sync_copy(hist_v, out_hbm.at[wid])


@functools.partial(
    pl.kernel,
    out_type=jax.ShapeDtypeStruct((NC, NP, H), jnp.float32),
    mesh=_mesh,
    compiler_params=_SC_PARAMS,
    scratch_types=(
        [pltpu.VMEM((K,), jnp.int32)] * NBUF +
        [pltpu.VMEM((K,), jnp.int32)] * NBUF +
        [pltpu.VMEM((K, H), jnp.float32)] * NBUF +
        [pltpu.VMEM_SHARED((NP, H), jnp.float32)] +
        [pltpu.SemaphoreType.DMA] * (4 * NBUF)
    ),
)
def _sc_scatter(xs_hbm, src_hbm, dst_hbm, zrows_hbm, out_hbm, *scr):
    c = lax.axis_index("c")
    s = lax.axis_index("s")
    wid = c * NS + s
    sidx = scr[0:NBUF]
    didx = scr[NBUF:2 * NBUF]
    rows = scr[2 * NBUF:3 * NBUF]
    acc_sh = scr[3 * NBUF]
    sems = scr[3 * NBUF + 1:]
    gsem = sems[0:NBUF]
    dsem = sems[NBUF:2 * NBUF]
    sisem = sems[2 * NBUF:3 * NBUF]
    ssem = sems[3 * NBUF:4 * NBUF]
    r0, r1 = rows[0], rows[1]

    pltpu.sync_copy(zrows_hbm, r0)
    for z in range(NRCH):
        pltpu.async_copy(
            r0, acc_sh.at[pl.ds(s * RPW + z * RCH, RCH)], gsem[0])
    for z in range(NRCH):
        pltpu.make_async_copy(
            r0, acc_sh.at[pl.ds(s * RPW, RCH)], gsem[0]).wait()
    plsc.subcore_barrier()

    for jj in range(NBUF):
        pltpu.async_copy(src_hbm.at[wid, jj], sidx[jj], sisem[jj])
    for jj in range(GL):
        pltpu.async_copy(dst_hbm.at[wid, jj], didx[jj], dsem[jj])
        pltpu.make_async_copy(src_hbm.at[wid, jj], sidx[jj], sisem[jj]).wait()
        pltpu.async_copy(xs_hbm.at[sidx[jj]], rows[jj], gsem[jj])

    def step(j, p):
        pg = (p + GL) % NBUF
        pltpu.make_async_copy(xs_hbm.at[sidx[p]], rows[p], gsem[p]).wait()

        @pl.when(j >= NBUF - GL)
        def _():
            pltpu.make_async_copy(
                rows[pg], acc_sh.at[didx[pg]], ssem[pg]).wait()

        pltpu.make_async_copy(dst_hbm.at[wid, j], didx[p], dsem[p]).wait()
        pltpu.async_copy(rows[p], acc_sh.at[didx[p]], ssem[p], add=True)

        @pl.when(j + NBUF < NCHUNK)
        def _():
            pltpu.async_copy(src_hbm.at[wid, j + NBUF], sidx[p], sisem[p])

        @pl.when(j + GL < NCHUNK)
        def _():
            pltpu.async_copy(dst_hbm.at[wid, j + GL], didx[pg], dsem[pg])
            pltpu.make_async_copy(
                src_hbm.at[wid, j + GL], sidx[pg], sisem[pg]).wait()
            pltpu.async_copy(xs_hbm.at[sidx[pg]], rows[pg], gsem[pg])

    def body(j, carry):
        for pp in range(NBUF):
            @pl.when(j % NBUF == pp)
            def _(pp=pp):
                step(j, pp)

        return carry

    lax.fori_loop(0, NCHUNK, body, 0)
    for jj in range(NCHUNK - (NBUF - GL), NCHUNK):
        pltpu.make_async_copy(
            rows[jj % NBUF], acc_sh.at[didx[jj % NBUF]], ssem[jj % NBUF]).wait()
    plsc.subcore_barrier()

    for z in range(NRCH):
        p = z & 1
        base = s * RPW + z * RCH
        if z >= 2:
            pltpu.make_async_copy(
                rows[p], out_hbm.at[c, pl.ds(base, RCH)], gsem[p]).wait()
        pltpu.sync_copy(acc_sh.at[pl.ds(base, RCH)], rows[p])
        pltpu.async_copy(rows[p], out_hbm.at[c, pl.ds(base, RCH)], gsem[p])
    pltpu.make_async_copy(
        r0, out_hbm.at[c, pl.ds(s * RPW, RCH)], gsem[0]).wait()
    pltpu.make_async_copy(
        r1, out_hbm.at[c, pl.ds(s * RPW, RCH)], gsem[1]).wait()


def _tc_pre_body(deg_ref, x_ref, w_ref, xs_ref, dis_ref):
    deg = jnp.sum(deg_ref[...], axis=0) + 1.0
    dis = lax.rsqrt(deg)
    xw = jnp.dot(x_ref[...], w_ref[...],
                 preferred_element_type=jnp.float32, precision=_HIGH)
    xs_ref[...] = xw * dis[:, None]
    dis_ref[...] = dis


def _tc_mid_body(p_ref, xs_ref, dis_ref, b_ref, w_ref, o_ref):
    dis = dis_ref[...]
    h = (p_ref[0] + p_ref[1] + xs_ref[...]) * dis[:, None] + b_ref[...]
    h = jnp.maximum(h, 0.0)
    o_ref[...] = jnp.dot(h, w_ref[...],
                         preferred_element_type=jnp.float32,
                         precision=_HIGH) * dis[:, None]


def _tc_head_body(p_ref, xs_ref, dis_ref, b_ref, w1_ref, b1_ref, w2_ref,
                  b2_ref, o_ref):
    dis = dis_ref[...]
    h = (p_ref[0] + p_ref[1] + xs_ref[...]) * dis[:, None] + b_ref[...]
    h = jnp.maximum(h, 0.0)
    h = jnp.dot(h, w1_ref[...], preferred_element_type=jnp.float32,
                precision=_HIGH) + b1_ref[...]
    h = jnp.maximum(h, 0.0)
    lg = jnp.dot(h, w2_ref[...], preferred_element_type=jnp.float32,
                 precision=_HIGH) + b2_ref[...]
    m = jnp.max(lg, axis=-1, keepdims=True)
    lg = lg - m
    o_ref[...] = lg - jnp.log(jnp.sum(jnp.exp(lg), axis=-1, keepdims=True))


def _rows(i):
    return (i, 0)


def _full2(i):
    return (0, 0)


def _full1(i):
    return (0,)


_tc_pre = pl.pallas_call(
    _tc_pre_body,
    grid=(GRID,),
    in_specs=[
        pl.BlockSpec((NW, BN), lambda i: (0, i)),
        pl.BlockSpec((BN, D), _rows),
        pl.BlockSpec((D, H), _full2),
    ],
    out_specs=[
        pl.BlockSpec((BN, H), _rows),
        pl.BlockSpec((BN,), lambda i: (i,)),
    ],
    out_shape=[
        jax.ShapeDtypeStruct((N, H), jnp.float32),
        jax.ShapeDtypeStruct((N,), jnp.float32),
    ],
)

_tc_mid = pl.pallas_call(
    _tc_mid_body,
    grid=(GRID,),
    in_specs=[
        pl.BlockSpec((NC, BN, H), lambda i: (0, i, 0)),
        pl.BlockSpec((BN, H), _rows),
        pl.BlockSpec((BN,), lambda i: (i,)),
        pl.BlockSpec((H,), _full1),
        pl.BlockSpec((H, H), _full2),
    ],
    out_specs=pl.BlockSpec((BN, H), _rows),
    out_shape=jax.ShapeDtypeStruct((N, H), jnp.float32),
)

_tc_head = pl.pallas_call(
    _tc_head_body,
    grid=(GRID,),
    in_specs=[
        pl.BlockSpec((NC, BN, H), lambda i: (0, i, 0)),
        pl.BlockSpec((BN, H), _rows),
        pl.BlockSpec((BN,), lambda i: (i,)),
        pl.BlockSpec((H,), _full1),
        pl.BlockSpec((H, H), _full2),
        pl.BlockSpec((H,), _full1),
        pl.BlockSpec((H, C), _full2),
        pl.BlockSpec((C,), _full1),
    ],
    out_specs=pl.BlockSpec((BN, C), _rows),
    out_shape=jax.ShapeDtypeStruct((N, C), jnp.float32),
)


def kernel(x, edge_index, W1, b1, W2, b2, Wl1, bl1, Wl2, bl2):
    src_r = edge_index[0].reshape(NW, NCHUNK, K)
    dst_r = edge_index[1].reshape(NW, NCHUNK, K)
    dst_f = edge_index[1].reshape(NW, EW)
    zeros_n = jnp.zeros((N,), jnp.float32)
    zeros_rows = jnp.zeros((RCH, H), jnp.float32)

    deg_part = _sc_degree(dst_f, zeros_n)
    xs1, dis = _tc_pre(deg_part, x, W1)
    part1 = _sc_scatter(xs1, src_r, dst_r, zeros_rows)
    xs2 = _tc_mid(part1, xs1, dis, b1, W2)
    part2 = _sc_scatter(xs2, src_r, dst_r, zeros_rows)
    return _tc_head(part2, xs2, dis, b2, Wl1, bl1, Wl2, bl2)
